# baseline (speedup 1.0000x reference)
import functools

import jax
import jax.numpy as jnp
from jax import lax
from jax.experimental import pallas as pl
from jax.experimental.pallas import tpu as pltpu
from jax.experimental.pallas import tpu_sc as plsc

POOL = 7
S, H, W, C = 4, 64, 64, 256
R = 128
NROI = S * R
NTAP = POOL * POOL * 9
NW = 32
ROI_PER_W = NROI // NW
CELLS_PER_W = ROI_PER_W * POOL * POOL
CHUNK = 8
NCHUNK = CELLS_PER_W // CHUNK
LANES = 16


def _tap_index_kernel(rois_ref, idx_ref):
    y = rois_ref[:, 0:1]
    x = rois_ref[:, 1:2]
    h = rois_ref[:, 2:3]
    w = rois_ref[:, 3:4]
    j = lax.broadcasted_iota(jnp.int32, (1, NTAP), 1)
    py = j // 63
    px = (j // 9) % 7
    dy = (j % 9) // 3
    dx = j % 3
    ystep = h.astype(jnp.float32) / float(POOL)
    xstep = w.astype(jnp.float32) / float(POOL)
    ystart = (py.astype(jnp.float32) * ystep).astype(jnp.int32)
    yend = jnp.where(py < POOL - 1,
                     ((py + 1).astype(jnp.float32) * ystep).astype(jnp.int32),
                     h)
    ysize = jnp.maximum(yend - ystart, 1)
    xstart = (px.astype(jnp.float32) * xstep).astype(jnp.int32)
    xend = jnp.where(px < POOL - 1,
                     ((px + 1).astype(jnp.float32) * xstep).astype(jnp.int32),
                     w)
    xsize = jnp.maximum(xend - xstart, 1)
    row = y + ystart + jnp.minimum(dy, ysize - 1)
    col = x + xstart + jnp.minimum(dx, xsize - 1)
    srow = lax.broadcasted_iota(jnp.int32, (NROI, 1), 0) // R
    idx_ref[...] = (srow * H + row) * W + col


def _tap_indices(x_rois):
    rois_flat = x_rois.reshape(NROI, 4)
    idx = pl.pallas_call(
        _tap_index_kernel,
        in_specs=[pl.BlockSpec((NROI, 4), lambda: (0, 0))],
        out_specs=pl.BlockSpec((NROI, NTAP), lambda: (0, 0)),
        out_shape=jax.ShapeDtypeStruct((NROI, NTAP), jnp.int32),
    )(rois_flat)
    return idx.reshape(NW, NCHUNK, CHUNK * 9)


def _roi_pool_sc(fmap_flat, idx):
    mesh = plsc.VectorSubcoreMesh(core_axis_name="c", subcore_axis_name="s")

    @functools.partial(
        pl.kernel,
        mesh=mesh,
        out_type=jax.ShapeDtypeStruct((NROI * POOL * POOL, C), jnp.float32),
        scratch_types=[
            pltpu.VMEM((NCHUNK, CHUNK * 9), jnp.int32),
            pltpu.VMEM((CHUNK * 9, C), jnp.float32),
            pltpu.VMEM((CHUNK, C), jnp.float32),
            pltpu.SemaphoreType.DMA,
        ],
    )
    def sc_kernel(fmap_hbm, idx_hbm, out_hbm, idx_v, rows_v, out_v, gsem):
        wid = lax.axis_index("s") * 2 + lax.axis_index("c")
        cell0 = wid * CELLS_PER_W
        pltpu.sync_copy(idx_hbm.at[wid], idx_v)

        def chunk_body(g, carry):
            pltpu.async_copy(fmap_hbm.at[idx_v.at[g]], rows_v, gsem).wait()

            def cell_body(cc, carry2):
                base = cc * 9
                for grp in range(C // LANES):
                    sl = pl.ds(grp * LANES, LANES)
                    v = rows_v[base, sl]
                    for t in range(1, 9):
                        v = jnp.maximum(v, rows_v[base + t, sl])
                    out_v[cc, sl] = v
                return carry2

            lax.fori_loop(0, CHUNK, cell_body, 0)
            pltpu.sync_copy(out_v, out_hbm.at[pl.ds(cell0 + g * CHUNK, CHUNK)])
            return carry

        lax.fori_loop(0, NCHUNK, chunk_body, 0)

    return sc_kernel(fmap_flat, idx)


@jax.jit
def kernel(x_maps, x_rois):
    idx = _tap_indices(x_rois)
    fmap_flat = x_maps.reshape(S * H * W, C)
    out = _roi_pool_sc(fmap_flat, idx)
    return out.reshape(S, R, POOL, POOL, C)


# speedup vs baseline: 1.5827x; 1.5827x over previous
import functools

import jax
import jax.numpy as jnp
from jax import lax
from jax.experimental import pallas as pl
from jax.experimental.pallas import tpu as pltpu
from jax.experimental.pallas import tpu_sc as plsc

POOL = 7
S, H, W, C = 4, 64, 64, 256
R = 128
NROI = S * R
NW = 32
ROI_PER_W = NROI // NW
CELLS_PER_W = ROI_PER_W * POOL * POOL
CHUNK = 8
IDXW = CHUNK * 9
NCHUNK = CELLS_PER_W // CHUNK
LANES = 16


NTAP = POOL * POOL * 9


def _tap_index_kernel(rois_ref, idx_ref):
    j = lax.broadcasted_iota(jnp.int32, (1, NTAP), 1)
    py = j // 63
    px = (j // 9) % 7
    dy = (j % 9) // 3
    dx = j % 3
    y = rois_ref[:, 0:1]
    x = rois_ref[:, 1:2]
    h = rois_ref[:, 2:3]
    w = rois_ref[:, 3:4]
    ystep = h.astype(jnp.float32) / float(POOL)
    xstep = w.astype(jnp.float32) / float(POOL)
    ystart = (py.astype(jnp.float32) * ystep).astype(jnp.int32)
    yend = jnp.where(py < POOL - 1,
                     ((py + 1).astype(jnp.float32) * ystep).astype(jnp.int32),
                     h)
    ysize = jnp.maximum(yend - ystart, 1)
    xstart = (px.astype(jnp.float32) * xstep).astype(jnp.int32)
    xend = jnp.where(px < POOL - 1,
                     ((px + 1).astype(jnp.float32) * xstep).astype(jnp.int32),
                     w)
    xsize = jnp.maximum(xend - xstart, 1)
    row = y + ystart + jnp.minimum(dy, ysize - 1)
    col = x + xstart + jnp.minimum(dx, xsize - 1)
    srow = lax.broadcasted_iota(jnp.int32, (NROI, 1), 0) // R
    idx_ref[...] = (srow * H + row) * W + col


def _tap_indices(x_rois):
    rois_flat = x_rois.reshape(NROI, 4)
    idx = pl.pallas_call(
        _tap_index_kernel,
        in_specs=[pl.BlockSpec((NROI, 4), lambda: (0, 0))],
        out_specs=pl.BlockSpec((NROI, NTAP), lambda: (0, 0)),
        out_shape=jax.ShapeDtypeStruct((NROI, NTAP), jnp.int32),
    )(rois_flat)
    return idx.reshape(NW, NCHUNK, IDXW)


def _roi_pool_sc(fmap_flat, idx):
    mesh = plsc.VectorSubcoreMesh(core_axis_name="c", subcore_axis_name="s")

    @functools.partial(
        pl.kernel,
        mesh=mesh,
        out_type=jax.ShapeDtypeStruct((NROI * POOL * POOL, C), jnp.float32),
        scratch_types=[
            pltpu.VMEM((NCHUNK, IDXW), jnp.int32),
            pltpu.VMEM((IDXW, C), jnp.float32),
            pltpu.VMEM((IDXW, C), jnp.float32),
            pltpu.VMEM((CHUNK, C), jnp.float32),
            pltpu.VMEM((CHUNK, C), jnp.float32),
            pltpu.SemaphoreType.DMA,
            pltpu.SemaphoreType.DMA,
            pltpu.SemaphoreType.DMA,
            pltpu.SemaphoreType.DMA,
        ],
    )
    def sc_kernel(fmap_hbm, idx_hbm, out_hbm, idx_v, rows0, rows1,
                  out0, out1, gsem0, gsem1, ssem0, ssem1):
        wid = lax.axis_index("s") * 2 + lax.axis_index("c")
        cell0 = wid * CELLS_PER_W
        pltpu.sync_copy(idx_hbm.at[wid], idx_v)

        def start_gather(g, rows_v, gsem):
            pltpu.make_async_copy(fmap_hbm.at[idx_v.at[g]], rows_v, gsem).start()

        def wait_gather(rows_v, gsem):
            pltpu.make_async_copy(fmap_hbm.at[pl.ds(0, IDXW)], rows_v, gsem).wait()

        def start_scatter(g, out_v, ssem):
            pltpu.make_async_copy(
                out_v, out_hbm.at[pl.ds(cell0 + g * CHUNK, CHUNK)], ssem).start()

        def wait_scatter(out_v, ssem):
            pltpu.make_async_copy(
                out_v, out_hbm.at[pl.ds(cell0, CHUNK)], ssem).wait()

        def compute(rows_v, out_v):
            def cell_body(cc, carry):
                base = cc * 9
                for grp in range(C // LANES):
                    sl = pl.ds(grp * LANES, LANES)
                    v = rows_v[base, sl]
                    for t in range(1, 9):
                        v = jnp.maximum(v, rows_v[base + t, sl])
                    out_v[cc, sl] = v
                return carry

            lax.fori_loop(0, CHUNK, cell_body, 0)

        start_gather(0, rows0, gsem0)
        start_gather(1, rows1, gsem1)

        def pair_body(k, carry):
            g0 = 2 * k
            wait_gather(rows0, gsem0)
            compute(rows0, out0)

            @pl.when(k < NCHUNK // 2 - 1)
            def _():
                start_gather(g0 + 2, rows0, gsem0)

            @pl.when(k > 0)
            def _():
                wait_scatter(out0, ssem0)

            start_scatter(g0, out0, ssem0)

            wait_gather(rows1, gsem1)
            compute(rows1, out1)

            @pl.when(k < NCHUNK // 2 - 1)
            def _():
                start_gather(g0 + 3, rows1, gsem1)

            @pl.when(k > 0)
            def _():
                wait_scatter(out1, ssem1)

            start_scatter(g0 + 1, out1, ssem1)
            return carry

        lax.fori_loop(0, NCHUNK // 2, pair_body, 0)
        wait_scatter(out0, ssem0)
        wait_scatter(out1, ssem1)

    return sc_kernel(fmap_flat, idx)


@jax.jit
def kernel(x_maps, x_rois):
    idx = _tap_indices(x_rois)
    fmap_flat = x_maps.reshape(S * H * W, C)
    out = _roi_pool_sc(fmap_flat, idx)
    return out.reshape(S, R, POOL, POOL, C)


# speedup vs baseline: 1.9784x; 1.2500x over previous
import functools

import jax
import jax.numpy as jnp
from jax import lax
from jax.experimental import pallas as pl
from jax.experimental.pallas import tpu as pltpu
from jax.experimental.pallas import tpu_sc as plsc

POOL = 7
S, H, W, C = 4, 64, 64, 256
R = 128
S_SC = 2
NROI_SC = S_SC * R
NTAP = POOL * POOL * 9
NW = 32
ROI_PER_W = NROI_SC // NW
CELLS_PER_W = ROI_PER_W * POOL * POOL
CHUNK = 8
IDXW = CHUNK * 9
NCHUNK = CELLS_PER_W // CHUNK
LANES = 16
WIN = 32
NEG_INF = float("-inf")


def _tap_index_kernel(rois_ref, idx_ref):
    j = lax.broadcasted_iota(jnp.int32, (1, NTAP), 1)
    py = j // 63
    px = (j // 9) % 7
    dy = (j % 9) // 3
    dx = j % 3
    y = rois_ref[:, 0:1]
    x = rois_ref[:, 1:2]
    h = rois_ref[:, 2:3]
    w = rois_ref[:, 3:4]
    ystep = h.astype(jnp.float32) / float(POOL)
    xstep = w.astype(jnp.float32) / float(POOL)
    ystart = (py.astype(jnp.float32) * ystep).astype(jnp.int32)
    yend = jnp.where(py < POOL - 1,
                     ((py + 1).astype(jnp.float32) * ystep).astype(jnp.int32),
                     h)
    ysize = jnp.maximum(yend - ystart, 1)
    xstart = (px.astype(jnp.float32) * xstep).astype(jnp.int32)
    xend = jnp.where(px < POOL - 1,
                     ((px + 1).astype(jnp.float32) * xstep).astype(jnp.int32),
                     w)
    xsize = jnp.maximum(xend - xstart, 1)
    row = y + ystart + jnp.minimum(dy, ysize - 1)
    col = x + xstart + jnp.minimum(dx, xsize - 1)
    srow = lax.broadcasted_iota(jnp.int32, (NROI_SC, 1), 0) // R
    idx_ref[...] = (srow * H + row) * W + col


def _tap_indices(rois_sc):
    idx = pl.pallas_call(
        _tap_index_kernel,
        in_specs=[pl.BlockSpec((NROI_SC, 4), lambda: (0, 0))],
        out_specs=pl.BlockSpec((NROI_SC, NTAP), lambda: (0, 0)),
        out_shape=jax.ShapeDtypeStruct((NROI_SC, NTAP), jnp.int32),
    )(rois_sc)
    return idx.reshape(NW, NCHUNK, IDXW)


def _roi_pool_sc(fmap_flat, idx):
    mesh = plsc.VectorSubcoreMesh(core_axis_name="c", subcore_axis_name="s")

    @functools.partial(
        pl.kernel,
        mesh=mesh,
        out_type=jax.ShapeDtypeStruct((NROI_SC * POOL * POOL, C), jnp.float32),
        scratch_types=[
            pltpu.VMEM((NCHUNK, IDXW), jnp.int32),
            pltpu.VMEM((IDXW, C), jnp.float32),
            pltpu.VMEM((IDXW, C), jnp.float32),
            pltpu.VMEM((CHUNK, C), jnp.float32),
            pltpu.VMEM((CHUNK, C), jnp.float32),
            pltpu.SemaphoreType.DMA,
            pltpu.SemaphoreType.DMA,
            pltpu.SemaphoreType.DMA,
            pltpu.SemaphoreType.DMA,
        ],
    )
    def sc_kernel(fmap_hbm, idx_hbm, out_hbm, idx_v, rows0, rows1,
                  out0, out1, gsem0, gsem1, ssem0, ssem1):
        wid = lax.axis_index("s") * 2 + lax.axis_index("c")
        cell0 = wid * CELLS_PER_W
        pltpu.sync_copy(idx_hbm.at[wid], idx_v)

        def start_gather(g, rows_v, gsem):
            pltpu.make_async_copy(fmap_hbm.at[idx_v.at[g]], rows_v, gsem).start()

        def wait_gather(rows_v, gsem):
            pltpu.make_async_copy(fmap_hbm.at[pl.ds(0, IDXW)], rows_v, gsem).wait()

        def start_scatter(g, out_v, ssem):
            pltpu.make_async_copy(
                out_v, out_hbm.at[pl.ds(cell0 + g * CHUNK, CHUNK)], ssem).start()

        def wait_scatter(out_v, ssem):
            pltpu.make_async_copy(
                out_v, out_hbm.at[pl.ds(cell0, CHUNK)], ssem).wait()

        def compute(rows_v, out_v):
            def cell_body(cc, carry):
                base = cc * 9
                for grp in range(C // LANES):
                    sl = pl.ds(grp * LANES, LANES)
                    v = rows_v[base, sl]
                    for t in range(1, 9):
                        v = jnp.maximum(v, rows_v[base + t, sl])
                    out_v[cc, sl] = v
                return carry

            lax.fori_loop(0, CHUNK, cell_body, 0)

        start_gather(0, rows0, gsem0)
        start_gather(1, rows1, gsem1)
        npair = NCHUNK // 2

        def pair_body(k, carry):
            g0 = 2 * k
            wait_gather(rows0, gsem0)
            compute(rows0, out0)
            start_gather(g0 + 2, rows0, gsem0)

            @pl.when(k > 0)
            def _():
                wait_scatter(out0, ssem0)

            start_scatter(g0, out0, ssem0)

            wait_gather(rows1, gsem1)
            compute(rows1, out1)

            @pl.when(k < npair - 1)
            def _():
                start_gather(g0 + 3, rows1, gsem1)

            @pl.when(k > 0)
            def _():
                wait_scatter(out1, ssem1)

            start_scatter(g0 + 1, out1, ssem1)
            return carry

        lax.fori_loop(0, npair, pair_body, 0)
        wait_gather(rows0, gsem0)
        compute(rows0, out0)
        wait_scatter(out0, ssem0)
        start_scatter(NCHUNK - 1, out0, ssem0)
        wait_scatter(out0, ssem0)
        wait_scatter(out1, ssem1)

    return sc_kernel(fmap_flat, idx)


def _roi_pool_tc_kernel(rois_ref, fmap_ref, out_ref, tmp_ref):
    r = pl.program_id(1)
    y = rois_ref[0, r, 0]
    x = rois_ref[0, r, 1]
    h = rois_ref[0, r, 2]
    w = rois_ref[0, r, 3]

    ystep = h.astype(jnp.float32) / float(POOL)
    xstep = w.astype(jnp.float32) / float(POOL)

    col0 = jnp.minimum((x // 8) * 8, W - WIN)
    col0 = pl.multiple_of(col0, 8)
    lx = x - col0

    for py in range(POOL):
        ystart = (jnp.float32(py) * ystep).astype(jnp.int32)
        if py + 1 < POOL:
            yend = (jnp.float32(py + 1) * ystep).astype(jnp.int32)
        else:
            yend = h
        ysize = jnp.maximum(yend - ystart, 1)
        rows = fmap_ref[0, pl.ds(y + ystart, 3), pl.ds(col0, WIN), :]
        dy = lax.broadcasted_iota(jnp.int32, (3, 1, 1), 0)
        rows = jnp.where(dy < ysize, rows, NEG_INF)
        tmp_ref[:, py, :] = jnp.max(rows, axis=0)

    for px in range(POOL):
        xstart = (jnp.float32(px) * xstep).astype(jnp.int32)
        if px + 1 < POOL:
            xend = (jnp.float32(px + 1) * xstep).astype(jnp.int32)
        else:
            xend = w
        xsize = jnp.maximum(xend - xstart, 1)
        cols = tmp_ref[pl.ds(lx + xstart, 3), :, :]
        dx = lax.broadcasted_iota(jnp.int32, (3, 1, 1), 0)
        cols = jnp.where(dx < xsize, cols, NEG_INF)
        out_ref[0, 0, :, px, :] = jnp.max(cols, axis=0)


def _roi_pool_tc(x_maps, x_rois):
    return pl.pallas_call(
        _roi_pool_tc_kernel,
        grid=(S - S_SC, R),
        in_specs=[
            pl.BlockSpec((1, R, 4), lambda s, r: (s + S_SC, 0, 0),
                         memory_space=pltpu.SMEM),
            pl.BlockSpec((1, H, W, C), lambda s, r: (s + S_SC, 0, 0, 0)),
        ],
        out_specs=pl.BlockSpec((1, 1, POOL, POOL, C),
                               lambda s, r: (s, r, 0, 0, 0)),
        out_shape=jax.ShapeDtypeStruct((S - S_SC, R, POOL, POOL, C),
                                       jnp.float32),
        scratch_shapes=[pltpu.VMEM((WIN, POOL, C), jnp.float32)],
    )(x_rois, x_maps)


@jax.jit
def kernel(x_maps, x_rois):
    idx = _tap_indices(x_rois[:S_SC].reshape(NROI_SC, 4))
    fmap_flat = x_maps.reshape(S * H * W, C)
    out_sc = _roi_pool_sc(fmap_flat, idx)
    out_tc = _roi_pool_tc(x_maps, x_rois)
    return jnp.concatenate(
        [out_sc.reshape(S_SC, R, POOL, POOL, C), out_tc], axis=0)


# speedup vs baseline: 2.2085x; 1.1163x over previous
import functools

import jax
import jax.numpy as jnp
from jax import lax
from jax.experimental import pallas as pl
from jax.experimental.pallas import tpu as pltpu
from jax.experimental.pallas import tpu_sc as plsc

POOL = 7
S, H, W, C = 4, 64, 64, 256
R = 128
NROI = S * R
NROI_SC = 320
NROI_TC = NROI - NROI_SC
NW = 32
ROI_PER_W = NROI_SC // NW
IDXROW = 512
CCHUNK = 7
IDXW = 64
NCHUNK = ROI_PER_W * POOL
CELLS_PER_W = ROI_PER_W * POOL * POOL
LANES = 16
WIN = 32
NEG_INF = float("-inf")


def _tap_index_kernel(rois_ref, idx_ref):
    j = lax.broadcasted_iota(jnp.int32, (1, IDXROW), 1)
    c = j // IDXW
    l = j % IDXW
    py = jnp.minimum(c, POOL - 1)
    px = jnp.minimum(l // 9, POOL - 1)
    t = l % 9
    dy = t // 3
    dx = t % 3
    y = rois_ref[:, 0:1]
    x = rois_ref[:, 1:2]
    h = rois_ref[:, 2:3]
    w = rois_ref[:, 3:4]
    ystep = h.astype(jnp.float32) / float(POOL)
    xstep = w.astype(jnp.float32) / float(POOL)
    ystart = (py.astype(jnp.float32) * ystep).astype(jnp.int32)
    yend = jnp.where(py < POOL - 1,
                     ((py + 1).astype(jnp.float32) * ystep).astype(jnp.int32),
                     h)
    ysize = jnp.maximum(yend - ystart, 1)
    xstart = (px.astype(jnp.float32) * xstep).astype(jnp.int32)
    xend = jnp.where(px < POOL - 1,
                     ((px + 1).astype(jnp.float32) * xstep).astype(jnp.int32),
                     w)
    xsize = jnp.maximum(xend - xstart, 1)
    row = y + ystart + jnp.minimum(dy, ysize - 1)
    col = x + xstart + jnp.minimum(dx, xsize - 1)
    srow = lax.broadcasted_iota(jnp.int32, (NROI_SC, 1), 0) // R
    idx_ref[...] = (srow * H + row) * W + col


def _tap_indices(rois_sc):
    idx = pl.pallas_call(
        _tap_index_kernel,
        in_specs=[pl.BlockSpec((NROI_SC, 4), lambda: (0, 0))],
        out_specs=pl.BlockSpec((NROI_SC, IDXROW), lambda: (0, 0)),
        out_shape=jax.ShapeDtypeStruct((NROI_SC, IDXROW), jnp.int32),
    )(rois_sc)
    return idx.reshape(NW, ROI_PER_W * IDXROW)


def _roi_pool_sc(fmap_flat, idx):
    mesh = plsc.VectorSubcoreMesh(core_axis_name="c", subcore_axis_name="s")

    @functools.partial(
        pl.kernel,
        mesh=mesh,
        out_type=jax.ShapeDtypeStruct((NW * NCHUNK, CCHUNK, C), jnp.float32),
        scratch_types=[
            pltpu.VMEM((ROI_PER_W * IDXROW,), jnp.int32),
            pltpu.VMEM((IDXW, C), jnp.float32),
            pltpu.VMEM((IDXW, C), jnp.float32),
            pltpu.VMEM((CCHUNK, C), jnp.float32),
            pltpu.VMEM((CCHUNK, C), jnp.float32),
            pltpu.SemaphoreType.DMA,
            pltpu.SemaphoreType.DMA,
            pltpu.SemaphoreType.DMA,
            pltpu.SemaphoreType.DMA,
        ],
    )
    def sc_kernel(fmap_hbm, idx_hbm, out_hbm, idx_v, rows0, rows1,
                  out0, out1, gsem0, gsem1, ssem0, ssem1):
        wid = lax.axis_index("s") * 2 + lax.axis_index("c")
        chunk0 = wid * NCHUNK
        pltpu.sync_copy(idx_hbm.at[wid], idx_v)

        def start_gather(g, rows_v, gsem):
            off = (g // POOL) * IDXROW + (g % POOL) * IDXW
            pltpu.make_async_copy(
                fmap_hbm.at[idx_v.at[pl.ds(off, IDXW)]], rows_v, gsem).start()

        def wait_gather(rows_v, gsem):
            pltpu.make_async_copy(fmap_hbm.at[pl.ds(0, IDXW)], rows_v, gsem).wait()

        def start_scatter(g, out_v, ssem):
            pltpu.make_async_copy(out_v, out_hbm.at[chunk0 + g], ssem).start()

        def wait_scatter(out_v, ssem):
            pltpu.make_async_copy(out_v, out_hbm.at[0], ssem).wait()

        def compute(rows_v, out_v):
            def cell_body(cc, carry):
                base = cc * 9
                for grp in range(C // LANES):
                    sl = pl.ds(grp * LANES, LANES)
                    v = rows_v[base, sl]
                    for t in range(1, 9):
                        v = jnp.maximum(v, rows_v[base + t, sl])
                    out_v[cc, sl] = v
                return carry

            lax.fori_loop(0, CCHUNK, cell_body, 0)

        start_gather(0, rows0, gsem0)
        start_gather(1, rows1, gsem1)
        npair = NCHUNK // 2

        def pair_body(k, carry):
            g0 = 2 * k
            wait_gather(rows0, gsem0)
            compute(rows0, out0)

            @pl.when(k < npair - 1)
            def _():
                start_gather(g0 + 2, rows0, gsem0)

            @pl.when(k > 0)
            def _():
                wait_scatter(out0, ssem0)

            start_scatter(g0, out0, ssem0)

            wait_gather(rows1, gsem1)
            compute(rows1, out1)

            @pl.when(k < npair - 1)
            def _():
                start_gather(g0 + 3, rows1, gsem1)

            @pl.when(k > 0)
            def _():
                wait_scatter(out1, ssem1)

            start_scatter(g0 + 1, out1, ssem1)
            return carry

        lax.fori_loop(0, npair, pair_body, 0)
        wait_scatter(out0, ssem0)
        wait_scatter(out1, ssem1)

    return sc_kernel(fmap_flat, idx)


def _roi_pool_tc_kernel(rois_ref, fmap_ref, out_ref, tmp_ref):
    r = pl.program_id(0) + NROI_SC
    y = rois_ref[r, 0]
    x = rois_ref[r, 1]
    h = rois_ref[r, 2]
    w = rois_ref[r, 3]

    ystep = h.astype(jnp.float32) / float(POOL)
    xstep = w.astype(jnp.float32) / float(POOL)

    col0 = jnp.minimum((x // 8) * 8, W - WIN)
    col0 = pl.multiple_of(col0, 8)
    lx = x - col0

    for py in range(POOL):
        ystart = (jnp.float32(py) * ystep).astype(jnp.int32)
        if py + 1 < POOL:
            yend = (jnp.float32(py + 1) * ystep).astype(jnp.int32)
        else:
            yend = h
        ysize = jnp.maximum(yend - ystart, 1)
        rows = fmap_ref[0, pl.ds(y + ystart, 3), pl.ds(col0, WIN), :]
        dy = lax.broadcasted_iota(jnp.int32, (3, 1, 1), 0)
        rows = jnp.where(dy < ysize, rows, NEG_INF)
        tmp_ref[:, py, :] = jnp.max(rows, axis=0)

    for px in range(POOL):
        xstart = (jnp.float32(px) * xstep).astype(jnp.int32)
        if px + 1 < POOL:
            xend = (jnp.float32(px + 1) * xstep).astype(jnp.int32)
        else:
            xend = w
        xsize = jnp.maximum(xend - xstart, 1)
        cols = tmp_ref[pl.ds(lx + xstart, 3), :, :]
        dx = lax.broadcasted_iota(jnp.int32, (3, 1, 1), 0)
        cols = jnp.where(dx < xsize, cols, NEG_INF)
        out_ref[0, :, px, :] = jnp.max(cols, axis=0)


def _roi_pool_tc(x_maps, rois_flat):
    return pl.pallas_call(
        _roi_pool_tc_kernel,
        grid=(NROI_TC,),
        in_specs=[
            pl.BlockSpec((NROI, 4), lambda r: (0, 0),
                         memory_space=pltpu.SMEM),
            pl.BlockSpec((1, H, W, C), lambda r: ((r + NROI_SC) // R, 0, 0, 0)),
        ],
        out_specs=pl.BlockSpec((1, POOL, POOL, C),
                               lambda r: (r + NROI_SC, 0, 0, 0)),
        out_shape=jax.ShapeDtypeStruct((NROI, POOL, POOL, C), jnp.float32),
        scratch_shapes=[pltpu.VMEM((WIN, POOL, C), jnp.float32)],
    )(rois_flat, x_maps)


@jax.jit
def kernel(x_maps, x_rois):
    rois_flat = x_rois.reshape(NROI, 4)
    idx = _tap_indices(rois_flat[:NROI_SC])
    fmap_flat = x_maps.reshape(S * H * W, C)
    out_sc = _roi_pool_sc(fmap_flat, idx)
    out_tc = _roi_pool_tc(x_maps, rois_flat)
    out = lax.dynamic_update_slice(
        out_tc, out_sc.reshape(NROI_SC, POOL, POOL, C), (0, 0, 0, 0))
    return out.reshape(S, R, POOL, POOL, C)


# speedup vs baseline: 2.3649x; 1.0708x over previous
import functools

import jax
import jax.numpy as jnp
from jax import lax
from jax.experimental import pallas as pl
from jax.experimental.pallas import tpu as pltpu
from jax.experimental.pallas import tpu_sc as plsc

POOL = 7
S, H, W, C = 4, 64, 64, 256
R = 128
NROI = S * R
NROI_SC = 288
NROI_TC = NROI - NROI_SC
NW = 32
ROI_PER_W = NROI_SC // NW
IDXROW = 512
CCHUNK = 7
IDXW = 64
NCHUNK = ROI_PER_W * POOL
CELLS_PER_W = ROI_PER_W * POOL * POOL
LANES = 16
WIN = 32
NEG_INF = float("-inf")


def _tap_index_kernel(rois_ref, idx_ref):
    j = lax.broadcasted_iota(jnp.int32, (1, IDXROW), 1)
    c = j // IDXW
    l = j % IDXW
    py = jnp.minimum(c, POOL - 1)
    px = jnp.minimum(l // 9, POOL - 1)
    t = l % 9
    dy = t // 3
    dx = t % 3
    y = rois_ref[:, 0:1]
    x = rois_ref[:, 1:2]
    h = rois_ref[:, 2:3]
    w = rois_ref[:, 3:4]
    ystep = h.astype(jnp.float32) / float(POOL)
    xstep = w.astype(jnp.float32) / float(POOL)
    ystart = (py.astype(jnp.float32) * ystep).astype(jnp.int32)
    yend = jnp.where(py < POOL - 1,
                     ((py + 1).astype(jnp.float32) * ystep).astype(jnp.int32),
                     h)
    ysize = jnp.maximum(yend - ystart, 1)
    xstart = (px.astype(jnp.float32) * xstep).astype(jnp.int32)
    xend = jnp.where(px < POOL - 1,
                     ((px + 1).astype(jnp.float32) * xstep).astype(jnp.int32),
                     w)
    xsize = jnp.maximum(xend - xstart, 1)
    row = y + ystart + jnp.minimum(dy, ysize - 1)
    col = x + xstart + jnp.minimum(dx, xsize - 1)
    srow = lax.broadcasted_iota(jnp.int32, (NROI_SC, 1), 0) // R
    idx_ref[...] = (srow * H + row) * W + col


def _tap_indices(rois_sc):
    idx = pl.pallas_call(
        _tap_index_kernel,
        in_specs=[pl.BlockSpec((NROI_SC, 4), lambda: (0, 0))],
        out_specs=pl.BlockSpec((NROI_SC, IDXROW), lambda: (0, 0)),
        out_shape=jax.ShapeDtypeStruct((NROI_SC, IDXROW), jnp.int32),
    )(rois_sc)
    return idx.reshape(NW, ROI_PER_W * IDXROW)


def _roi_pool_sc(fmap_flat, idx):
    mesh = plsc.VectorSubcoreMesh(core_axis_name="c", subcore_axis_name="s")

    @functools.partial(
        pl.kernel,
        mesh=mesh,
        out_type=jax.ShapeDtypeStruct((NW * NCHUNK, CCHUNK, C), jnp.float32),
        scratch_types=[
            pltpu.VMEM((ROI_PER_W * IDXROW,), jnp.int32),
            pltpu.VMEM((IDXW, C), jnp.float32),
            pltpu.VMEM((IDXW, C), jnp.float32),
            pltpu.VMEM((CCHUNK, C), jnp.float32),
            pltpu.VMEM((CCHUNK, C), jnp.float32),
            pltpu.SemaphoreType.DMA,
            pltpu.SemaphoreType.DMA,
            pltpu.SemaphoreType.DMA,
            pltpu.SemaphoreType.DMA,
        ],
    )
    def sc_kernel(fmap_hbm, idx_hbm, out_hbm, idx_v, rows0, rows1,
                  out0, out1, gsem0, gsem1, ssem0, ssem1):
        wid = lax.axis_index("s") * 2 + lax.axis_index("c")
        chunk0 = wid * NCHUNK
        pltpu.sync_copy(idx_hbm.at[wid], idx_v)

        def start_gather(g, rows_v, gsem):
            off = (g // POOL) * IDXROW + (g % POOL) * IDXW
            pltpu.make_async_copy(
                fmap_hbm.at[idx_v.at[pl.ds(off, IDXW)]], rows_v, gsem).start()

        def wait_gather(rows_v, gsem):
            pltpu.make_async_copy(fmap_hbm.at[pl.ds(0, IDXW)], rows_v, gsem).wait()

        def start_scatter(g, out_v, ssem):
            pltpu.make_async_copy(out_v, out_hbm.at[chunk0 + g], ssem).start()

        def wait_scatter(out_v, ssem):
            pltpu.make_async_copy(out_v, out_hbm.at[0], ssem).wait()

        def compute(rows_v, out_v):
            def cell_body(cc, carry):
                base = cc * 9
                for grp in range(C // LANES):
                    sl = pl.ds(grp * LANES, LANES)
                    v = rows_v[base, sl]
                    for t in range(1, 9):
                        v = jnp.maximum(v, rows_v[base + t, sl])
                    out_v[cc, sl] = v
                return carry

            lax.fori_loop(0, CCHUNK, cell_body, 0)

        start_gather(0, rows0, gsem0)
        start_gather(1, rows1, gsem1)
        npair = NCHUNK // 2
        odd = NCHUNK % 2 == 1

        def pair_body(k, carry):
            g0 = 2 * k
            wait_gather(rows0, gsem0)
            compute(rows0, out0)

            if odd:
                start_gather(g0 + 2, rows0, gsem0)
            else:
                @pl.when(k < npair - 1)
                def _():
                    start_gather(g0 + 2, rows0, gsem0)

            @pl.when(k > 0)
            def _():
                wait_scatter(out0, ssem0)

            start_scatter(g0, out0, ssem0)

            wait_gather(rows1, gsem1)
            compute(rows1, out1)

            @pl.when(k < npair - 1)
            def _():
                start_gather(g0 + 3, rows1, gsem1)

            @pl.when(k > 0)
            def _():
                wait_scatter(out1, ssem1)

            start_scatter(g0 + 1, out1, ssem1)
            return carry

        lax.fori_loop(0, npair, pair_body, 0)
        if odd:
            wait_gather(rows0, gsem0)
            compute(rows0, out0)
            wait_scatter(out0, ssem0)
            start_scatter(NCHUNK - 1, out0, ssem0)
        wait_scatter(out0, ssem0)
        wait_scatter(out1, ssem1)

    return sc_kernel(fmap_flat, idx)


def _roi_pool_tc_kernel(rois_ref, fmap_ref, out_ref, tmp_ref):
    r = pl.program_id(0) + NROI_SC
    y = rois_ref[r, 0]
    x = rois_ref[r, 1]
    h = rois_ref[r, 2]
    w = rois_ref[r, 3]

    ystep = h.astype(jnp.float32) / float(POOL)
    xstep = w.astype(jnp.float32) / float(POOL)

    col0 = jnp.minimum((x // 8) * 8, W - WIN)
    col0 = pl.multiple_of(col0, 8)
    lx = x - col0

    for py in range(POOL):
        ystart = (jnp.float32(py) * ystep).astype(jnp.int32)
        if py + 1 < POOL:
            yend = (jnp.float32(py + 1) * ystep).astype(jnp.int32)
        else:
            yend = h
        ysize = jnp.maximum(yend - ystart, 1)
        rows = fmap_ref[0, pl.ds(y + ystart, 3), pl.ds(col0, WIN), :]
        dy = lax.broadcasted_iota(jnp.int32, (3, 1, 1), 0)
        rows = jnp.where(dy < ysize, rows, NEG_INF)
        tmp_ref[:, py, :] = jnp.max(rows, axis=0)

    for px in range(POOL):
        xstart = (jnp.float32(px) * xstep).astype(jnp.int32)
        if px + 1 < POOL:
            xend = (jnp.float32(px + 1) * xstep).astype(jnp.int32)
        else:
            xend = w
        xsize = jnp.maximum(xend - xstart, 1)
        cols = tmp_ref[pl.ds(lx + xstart, 3), :, :]
        dx = lax.broadcasted_iota(jnp.int32, (3, 1, 1), 0)
        cols = jnp.where(dx < xsize, cols, NEG_INF)
        out_ref[:, px, :] = jnp.max(cols, axis=0)


def _roi_pool_tc(x_maps, rois_flat):
    return pl.pallas_call(
        _roi_pool_tc_kernel,
        grid=(NROI_TC,),
        in_specs=[
            pl.BlockSpec((NROI, 4), lambda r: (0, 0),
                         memory_space=pltpu.SMEM),
            pl.BlockSpec((1, H, W, C), lambda r: ((r + NROI_SC) // R, 0, 0, 0)),
        ],
        out_specs=pl.BlockSpec((POOL, POOL, C),
                               lambda r: (r + NROI_SC, 0, 0)),
        out_shape=jax.ShapeDtypeStruct((NROI * POOL, POOL, C), jnp.float32),
        scratch_shapes=[pltpu.VMEM((WIN, POOL, C), jnp.float32)],
    )(rois_flat, x_maps)


@jax.jit
def kernel(x_maps, x_rois):
    rois_flat = x_rois.reshape(NROI, 4)
    idx = _tap_indices(rois_flat[:NROI_SC])
    fmap_flat = x_maps.reshape(S * H * W, C)
    out_sc = _roi_pool_sc(fmap_flat, idx)
    out_tc = _roi_pool_tc(x_maps, rois_flat)
    out = lax.dynamic_update_slice(out_tc, out_sc, (0, 0, 0))
    return out.reshape(S, R, POOL, POOL, C)


# speedup vs baseline: 2.6942x; 1.1392x over previous
import functools

import jax
import jax.numpy as jnp
from jax import lax
from jax.experimental import pallas as pl
from jax.experimental.pallas import tpu as pltpu
from jax.experimental.pallas import tpu_sc as plsc

POOL = 7
S, H, W, C = 4, 64, 64, 256
R = 128
NROI = S * R
S_SC = 2
NROI_SC = S_SC * R
NROI_TC = NROI - NROI_SC
NW = 32
ROI_PER_W = NROI_SC // NW
IDXROW = 512
IDXW = 64
NCHUNK = ROI_PER_W * POOL
LANES = 16
TCB = 8
WIN = 32
NEG_INF = float("-inf")


def _tap_index_kernel(rois_ref, idx_ref):
    j = lax.broadcasted_iota(jnp.int32, (1, IDXROW), 1)
    c = j // IDXW
    l = j % IDXW
    py = jnp.minimum(c, POOL - 1)
    px = jnp.minimum(l // 9, POOL - 1)
    t = l % 9
    dy = t // 3
    dx = t % 3
    y = rois_ref[:, 0:1]
    x = rois_ref[:, 1:2]
    h = rois_ref[:, 2:3]
    w = rois_ref[:, 3:4]
    ystep = h.astype(jnp.float32) / float(POOL)
    xstep = w.astype(jnp.float32) / float(POOL)
    ystart = (py.astype(jnp.float32) * ystep).astype(jnp.int32)
    yend = jnp.where(py < POOL - 1,
                     ((py + 1).astype(jnp.float32) * ystep).astype(jnp.int32),
                     h)
    ysize = jnp.maximum(yend - ystart, 1)
    xstart = (px.astype(jnp.float32) * xstep).astype(jnp.int32)
    xend = jnp.where(px < POOL - 1,
                     ((px + 1).astype(jnp.float32) * xstep).astype(jnp.int32),
                     w)
    xsize = jnp.maximum(xend - xstart, 1)
    row = y + ystart + jnp.minimum(dy, ysize - 1)
    col = x + xstart + jnp.minimum(dx, xsize - 1)
    srow = lax.broadcasted_iota(jnp.int32, (NROI_SC, 1), 0) // R
    idx_ref[...] = (srow * H + row) * W + col


def _tap_indices(rois_sc):
    idx = pl.pallas_call(
        _tap_index_kernel,
        in_specs=[pl.BlockSpec((NROI_SC, 4), lambda: (0, 0))],
        out_specs=pl.BlockSpec((NROI_SC, IDXROW), lambda: (0, 0)),
        out_shape=jax.ShapeDtypeStruct((NROI_SC, IDXROW), jnp.int32),
    )(rois_sc)
    return idx.reshape(NW, ROI_PER_W * IDXROW)


def _roi_pool_sc(fmap_flat, idx):
    mesh = plsc.VectorSubcoreMesh(core_axis_name="c", subcore_axis_name="s")

    @functools.partial(
        pl.kernel,
        mesh=mesh,
        out_type=jax.ShapeDtypeStruct((S_SC, POOL, POOL, R, C), jnp.float32),
        scratch_types=[
            pltpu.VMEM((ROI_PER_W * IDXROW,), jnp.int32),
            pltpu.VMEM((IDXW, C), jnp.float32),
            pltpu.VMEM((IDXW, C), jnp.float32),
            pltpu.VMEM((2, POOL, ROI_PER_W, C), jnp.float32),
            pltpu.SemaphoreType.DMA,
            pltpu.SemaphoreType.DMA,
            pltpu.SemaphoreType.DMA,
            pltpu.SemaphoreType.DMA,
        ],
    )
    def sc_kernel(fmap_hbm, idx_hbm, out_hbm, idx_v, rows0, rows1,
                  out_v, gsem0, gsem1, ssem0, ssem1):
        wid = lax.axis_index("s") * 2 + lax.axis_index("c")
        smp = wid // 16
        r0 = pl.multiple_of((wid % 16) * ROI_PER_W, 8)
        pltpu.sync_copy(idx_hbm.at[wid], idx_v)

        def start_gather(g, rows_v, gsem):
            off = (g % ROI_PER_W) * IDXROW + (g // ROI_PER_W) * IDXW
            pltpu.make_async_copy(
                fmap_hbm.at[idx_v.at[pl.ds(off, IDXW)]], rows_v, gsem).start()

        def wait_gather(rows_v, gsem):
            pltpu.make_async_copy(fmap_hbm.at[pl.ds(0, IDXW)], rows_v, gsem).wait()

        def start_scatter(py, pyb, ssem):
            pltpu.make_async_copy(
                out_v.at[pyb], out_hbm.at[smp, py, :, pl.ds(r0, ROI_PER_W)],
                ssem).start()

        def wait_scatter(ssem):
            pltpu.make_async_copy(
                out_v.at[0], out_hbm.at[0, 0, :, pl.ds(0, ROI_PER_W)],
                ssem).wait()

        def compute(pyb, rr, rows_v):
            def cell_body(px, carry):
                base = px * 9
                for grp in range(C // LANES):
                    sl = pl.ds(grp * LANES, LANES)
                    v = rows_v[base, sl]
                    for t in range(1, 9):
                        v = jnp.maximum(v, rows_v[base + t, sl])
                    out_v[pyb, px, rr, sl] = v
                return carry

            lax.fori_loop(0, POOL, cell_body, 0)

        start_gather(0, rows0, gsem0)
        start_gather(1, rows1, gsem1)

        def py_body(py, carry):
            pyb = py % 2

            @pl.when((py >= 2) & (pyb == 0))
            def _():
                wait_scatter(ssem0)

            @pl.when((py >= 2) & (pyb == 1))
            def _():
                wait_scatter(ssem1)

            for rr in range(ROI_PER_W):
                g = py * ROI_PER_W + rr
                rows_v, gsem = (rows0, gsem0) if rr % 2 == 0 else (rows1, gsem1)
                wait_gather(rows_v, gsem)
                compute(pyb, rr, rows_v)
                if rr < ROI_PER_W - 2:
                    start_gather(g + 2, rows_v, gsem)
                else:
                    @pl.when(py < POOL - 1)
                    def _():
                        start_gather(g + 2, rows_v, gsem)

            @pl.when(pyb == 0)
            def _():
                start_scatter(py, 0, ssem0)

            @pl.when(pyb == 1)
            def _():
                start_scatter(py, 1, ssem1)

            return carry

        lax.fori_loop(0, POOL, py_body, 0)
        wait_scatter(ssem0)
        wait_scatter(ssem1)

    return sc_kernel(fmap_flat, idx)


def _roi_pool_tc_kernel(rois_ref, fmap_ref, out_ref, tmp_ref):
    pg = pl.program_id(0)
    for rr in range(TCB):
        rg = NROI_SC + pg * TCB + rr
        y = rois_ref[rg, 0]
        x = rois_ref[rg, 1]
        h = rois_ref[rg, 2]
        w = rois_ref[rg, 3]

        ystep = h.astype(jnp.float32) / float(POOL)
        xstep = w.astype(jnp.float32) / float(POOL)

        col0 = jnp.minimum((x // 8) * 8, W - WIN)
        col0 = pl.multiple_of(col0, 8)
        lx = x - col0

        for py in range(POOL):
            ystart = (jnp.float32(py) * ystep).astype(jnp.int32)
            if py + 1 < POOL:
                yend = (jnp.float32(py + 1) * ystep).astype(jnp.int32)
            else:
                yend = h
            ysize = jnp.maximum(yend - ystart, 1)
            rows = fmap_ref[0, pl.ds(y + ystart, 3), pl.ds(col0, WIN), :]
            dy = lax.broadcasted_iota(jnp.int32, (3, 1, 1), 0)
            rows = jnp.where(dy < ysize, rows, NEG_INF)
            tmp_ref[:, py, :] = jnp.max(rows, axis=0)

        for px in range(POOL):
            xstart = (jnp.float32(px) * xstep).astype(jnp.int32)
            if px + 1 < POOL:
                xend = (jnp.float32(px + 1) * xstep).astype(jnp.int32)
            else:
                xend = w
            xsize = jnp.maximum(xend - xstart, 1)
            cols = tmp_ref[pl.ds(lx + xstart, 3), :, :]
            dx = lax.broadcasted_iota(jnp.int32, (3, 1, 1), 0)
            cols = jnp.where(dx < xsize, cols, NEG_INF)
            out_ref[0, :, px, rr, :] = jnp.max(cols, axis=0)


def _roi_pool_tc(x_maps, rois_flat):
    ngroups = NROI_TC // TCB
    gper = R // TCB
    return pl.pallas_call(
        _roi_pool_tc_kernel,
        grid=(ngroups,),
        in_specs=[
            pl.BlockSpec((NROI, 4), lambda g: (0, 0),
                         memory_space=pltpu.SMEM),
            pl.BlockSpec((1, H, W, C), lambda g: (S_SC + g // gper, 0, 0, 0)),
        ],
        out_specs=pl.BlockSpec((1, POOL, POOL, TCB, C),
                               lambda g: (S_SC + g // gper, 0, 0, g % gper, 0)),
        out_shape=jax.ShapeDtypeStruct((S, POOL, POOL, R, C), jnp.float32),
        scratch_shapes=[pltpu.VMEM((WIN, POOL, C), jnp.float32)],
    )(rois_flat, x_maps)


@jax.jit
def kernel(x_maps, x_rois):
    rois_flat = x_rois.reshape(NROI, 4)
    idx = _tap_indices(rois_flat[:NROI_SC])
    fmap_flat = x_maps.reshape(S * H * W, C)
    out_sc = _roi_pool_sc(fmap_flat, idx)
    out_tc = _roi_pool_tc(x_maps, rois_flat)
    out = lax.dynamic_update_slice(out_tc, out_sc, (0, 0, 0, 0, 0))
    return out.transpose(0, 3, 1, 2, 4)


# speedup vs baseline: 3.1496x; 1.1690x over previous
import functools

import jax
import jax.numpy as jnp
from jax import lax
from jax.experimental import pallas as pl
from jax.experimental.pallas import tpu as pltpu
from jax.experimental.pallas import tpu_sc as plsc

POOL = 7
S, H, W, C = 4, 64, 64, 256
R = 128
NROI = S * R
NROI_SC = 192
NROI_TC = NROI - NROI_SC
NW = 32
NBLK = NROI_SC // 8
NGRP = NBLK * POOL
IDXROW = 512
IDXW = 64
LANES = 16
TCB = 8
WIN = 32
NEG_INF = float("-inf")


def _tap_index_kernel(rois_ref, idx_ref):
    j = lax.broadcasted_iota(jnp.int32, (1, IDXROW), 1)
    c = j // IDXW
    l = j % IDXW
    py = jnp.minimum(c, POOL - 1)
    px = jnp.minimum(l // 9, POOL - 1)
    t = l % 9
    dy = t // 3
    dx = t % 3
    y = rois_ref[:, 0:1]
    x = rois_ref[:, 1:2]
    h = rois_ref[:, 2:3]
    w = rois_ref[:, 3:4]
    ystep = h.astype(jnp.float32) / float(POOL)
    xstep = w.astype(jnp.float32) / float(POOL)
    ystart = (py.astype(jnp.float32) * ystep).astype(jnp.int32)
    yend = jnp.where(py < POOL - 1,
                     ((py + 1).astype(jnp.float32) * ystep).astype(jnp.int32),
                     h)
    ysize = jnp.maximum(yend - ystart, 1)
    xstart = (px.astype(jnp.float32) * xstep).astype(jnp.int32)
    xend = jnp.where(px < POOL - 1,
                     ((px + 1).astype(jnp.float32) * xstep).astype(jnp.int32),
                     w)
    xsize = jnp.maximum(xend - xstart, 1)
    row = y + ystart + jnp.minimum(dy, ysize - 1)
    col = x + xstart + jnp.minimum(dx, xsize - 1)
    srow = lax.broadcasted_iota(jnp.int32, (NROI_SC, 1), 0) // R
    idx_ref[...] = (srow * H + row) * W + col


def _tap_indices(rois_sc):
    idx = pl.pallas_call(
        _tap_index_kernel,
        in_specs=[pl.BlockSpec((NROI_SC, 4), lambda: (0, 0))],
        out_specs=pl.BlockSpec((NROI_SC, IDXROW), lambda: (0, 0)),
        out_shape=jax.ShapeDtypeStruct((NROI_SC, IDXROW), jnp.int32),
    )(rois_sc)
    return idx


def _roi_pool_sc(fmap_flat, idx):
    mesh = plsc.VectorSubcoreMesh(core_axis_name="c", subcore_axis_name="s")

    @functools.partial(
        pl.kernel,
        mesh=mesh,
        out_type=jax.ShapeDtypeStruct((2, POOL, POOL, R, C), jnp.float32),
        scratch_types=[
            pltpu.VMEM((16, IDXROW), jnp.int32),
            pltpu.VMEM((IDXW, C), jnp.float32),
            pltpu.VMEM((IDXW, C), jnp.float32),
            pltpu.VMEM((2, POOL, 8, C), jnp.float32),
            pltpu.SemaphoreType.DMA,
            pltpu.SemaphoreType.DMA,
            pltpu.SemaphoreType.DMA,
            pltpu.SemaphoreType.DMA,
        ],
    )
    def sc_kernel(fmap_hbm, idx_hbm, out_hbm, idx_v, rows0, rows1,
                  out_v, gsem0, gsem1, ssem0, ssem1):
        wid = lax.axis_index("s") * 2 + lax.axis_index("c")
        q0 = (NGRP * wid) // NW
        q1 = (NGRP * (wid + 1)) // NW
        nq = q1 - q0
        bst = jnp.minimum(q0 // POOL, NBLK - 2)
        pltpu.sync_copy(idx_hbm.at[pl.ds(bst * 8, 16)], idx_v)

        def start_gather(q, rr, rows_v, gsem):
            lrow = (q // POOL - bst) * 8 + rr
            coff = (q % POOL) * IDXW
            pltpu.make_async_copy(
                fmap_hbm.at[idx_v.at[lrow, pl.ds(coff, IDXW)]],
                rows_v, gsem).start()

        def wait_gather(rows_v, gsem):
            pltpu.make_async_copy(fmap_hbm.at[pl.ds(0, IDXW)], rows_v, gsem).wait()

        def start_scatter(q, sb, ssem):
            b = q // POOL
            py = q % POOL
            smp = b // 16
            r0 = pl.multiple_of((b % 16) * 8, 8)
            pltpu.make_async_copy(
                out_v.at[sb], out_hbm.at[smp, py, :, pl.ds(r0, 8)],
                ssem).start()

        def wait_scatter(ssem):
            pltpu.make_async_copy(
                out_v.at[0], out_hbm.at[0, 0, :, pl.ds(0, 8)],
                ssem).wait()

        def compute(sb, rr, rows_v):
            def cell_body(px, carry):
                base = px * 9
                for grp in range(C // LANES):
                    sl = pl.ds(grp * LANES, LANES)
                    v = rows_v[base, sl]
                    for t in range(1, 9):
                        v = jnp.maximum(v, rows_v[base + t, sl])
                    out_v[sb, px, rr, sl] = v
                return carry

            lax.fori_loop(0, POOL, cell_body, 0)

        start_gather(q0, 0, rows0, gsem0)
        start_gather(q0, 1, rows1, gsem1)

        def group_body(i, carry):
            q = q0 + i
            sb = i % 2

            @pl.when((i >= 2) & (sb == 0))
            def _():
                wait_scatter(ssem0)

            @pl.when((i >= 2) & (sb == 1))
            def _():
                wait_scatter(ssem1)

            for rr in range(8):
                rows_v, gsem = (rows0, gsem0) if rr % 2 == 0 else (rows1, gsem1)
                wait_gather(rows_v, gsem)
                compute(sb, rr, rows_v)
                if rr < 6:
                    start_gather(q, rr + 2, rows_v, gsem)
                else:
                    @pl.when(i < nq - 1)
                    def _():
                        start_gather(q + 1, rr - 6, rows_v, gsem)

            @pl.when(sb == 0)
            def _():
                start_scatter(q, 0, ssem0)

            @pl.when(sb == 1)
            def _():
                start_scatter(q, 1, ssem1)

            return carry

        lax.fori_loop(0, nq, group_body, 0)
        wait_scatter(ssem0)
        wait_scatter(ssem1)

    return sc_kernel(fmap_flat, idx)


def _roi_pool_tc_kernel(rois_ref, fmap_ref, out_ref, tmp_ref):
    pg = pl.program_id(0)
    for rr in range(TCB):
        rg = NROI_SC + pg * TCB + rr
        y = rois_ref[rg, 0]
        x = rois_ref[rg, 1]
        h = rois_ref[rg, 2]
        w = rois_ref[rg, 3]

        ystep = h.astype(jnp.float32) / float(POOL)
        xstep = w.astype(jnp.float32) / float(POOL)

        col0 = jnp.minimum((x // 8) * 8, W - WIN)
        col0 = pl.multiple_of(col0, 8)
        lx = x - col0

        for py in range(POOL):
            ystart = (jnp.float32(py) * ystep).astype(jnp.int32)
            if py + 1 < POOL:
                yend = (jnp.float32(py + 1) * ystep).astype(jnp.int32)
            else:
                yend = h
            ysize = jnp.maximum(yend - ystart, 1)
            rows = fmap_ref[0, pl.ds(y + ystart, 3), pl.ds(col0, WIN), :]
            dy = lax.broadcasted_iota(jnp.int32, (3, 1, 1), 0)
            rows = jnp.where(dy < ysize, rows, NEG_INF)
            tmp_ref[:, py, :] = jnp.max(rows, axis=0)

        for px in range(POOL):
            xstart = (jnp.float32(px) * xstep).astype(jnp.int32)
            if px + 1 < POOL:
                xend = (jnp.float32(px + 1) * xstep).astype(jnp.int32)
            else:
                xend = w
            xsize = jnp.maximum(xend - xstart, 1)
            cols = tmp_ref[pl.ds(lx + xstart, 3), :, :]
            dx = lax.broadcasted_iota(jnp.int32, (3, 1, 1), 0)
            cols = jnp.where(dx < xsize, cols, NEG_INF)
            out_ref[0, :, px, rr, :] = jnp.max(cols, axis=0)


def _roi_pool_tc(x_maps, rois_flat):
    ngroups = NROI_TC // TCB
    goff = NROI_SC // TCB
    return pl.pallas_call(
        _roi_pool_tc_kernel,
        grid=(ngroups,),
        in_specs=[
            pl.BlockSpec((NROI, 4), lambda g: (0, 0),
                         memory_space=pltpu.SMEM),
            pl.BlockSpec((1, H, W, C), lambda g: ((goff + g) // 16, 0, 0, 0)),
        ],
        out_specs=pl.BlockSpec((1, POOL, POOL, TCB, C),
                               lambda g: ((goff + g) // 16, 0, 0,
                                          (goff + g) % 16, 0)),
        out_shape=jax.ShapeDtypeStruct((S, POOL, POOL, R, C), jnp.float32),
        scratch_shapes=[pltpu.VMEM((WIN, POOL, C), jnp.float32)],
    )(rois_flat, x_maps)


@jax.jit
def kernel(x_maps, x_rois):
    rois_flat = x_rois.reshape(NROI, 4)
    idx = _tap_indices(rois_flat[:NROI_SC])
    fmap_flat = x_maps.reshape(S * H * W, C)
    out_sc = _roi_pool_sc(fmap_flat, idx)
    out_tc = _roi_pool_tc(x_maps, rois_flat)
    out = lax.dynamic_update_slice(out_tc, out_sc[0:1], (0, 0, 0, 0, 0))
    out = lax.dynamic_update_slice(out, out_sc[1:2, :, :, :R // 2],
                                   (1, 0, 0, 0, 0))
    return out.transpose(0, 3, 1, 2, 4)


# speedup vs baseline: 3.3007x; 1.0480x over previous
import functools

import jax
import jax.numpy as jnp
from jax import lax
from jax.experimental import pallas as pl
from jax.experimental.pallas import tpu as pltpu
from jax.experimental.pallas import tpu_sc as plsc

POOL = 7
S, H, W, C = 4, 64, 64, 256
R = 128
NROI = S * R
NROI_SC = 176
NROI_TC = NROI - NROI_SC
NW = 32
NBLK = NROI_SC // 8
NGRP = NBLK * POOL
IDXROW = 512
IDXW = 64
LANES = 16
TCB = 8
WIN = 32
NEG_INF = float("-inf")


def _tap_index_kernel(rois_ref, idx_ref):
    j = lax.broadcasted_iota(jnp.int32, (1, IDXROW), 1)
    c = j // IDXW
    l = j % IDXW
    py = jnp.minimum(c, POOL - 1)
    px = jnp.minimum(l // 9, POOL - 1)
    t = l % 9
    dy = t // 3
    dx = t % 3
    y = rois_ref[:, 0:1]
    x = rois_ref[:, 1:2]
    h = rois_ref[:, 2:3]
    w = rois_ref[:, 3:4]
    ystep = h.astype(jnp.float32) / float(POOL)
    xstep = w.astype(jnp.float32) / float(POOL)
    ystart = (py.astype(jnp.float32) * ystep).astype(jnp.int32)
    yend = jnp.where(py < POOL - 1,
                     ((py + 1).astype(jnp.float32) * ystep).astype(jnp.int32),
                     h)
    ysize = jnp.maximum(yend - ystart, 1)
    xstart = (px.astype(jnp.float32) * xstep).astype(jnp.int32)
    xend = jnp.where(px < POOL - 1,
                     ((px + 1).astype(jnp.float32) * xstep).astype(jnp.int32),
                     w)
    xsize = jnp.maximum(xend - xstart, 1)
    row = y + ystart + jnp.minimum(dy, ysize - 1)
    col = x + xstart + jnp.minimum(dx, xsize - 1)
    srow = lax.broadcasted_iota(jnp.int32, (NROI_SC, 1), 0) // R
    idx_ref[...] = (srow * H + row) * W + col


def _tap_indices(rois_sc):
    idx = pl.pallas_call(
        _tap_index_kernel,
        in_specs=[pl.BlockSpec((NROI_SC, 4), lambda: (0, 0))],
        out_specs=pl.BlockSpec((NROI_SC, IDXROW), lambda: (0, 0)),
        out_shape=jax.ShapeDtypeStruct((NROI_SC, IDXROW), jnp.int32),
    )(rois_sc)
    return idx


def _roi_pool_sc(fmap_flat, idx):
    mesh = plsc.VectorSubcoreMesh(core_axis_name="c", subcore_axis_name="s")

    @functools.partial(
        pl.kernel,
        mesh=mesh,
        out_type=jax.ShapeDtypeStruct((2, POOL, POOL, R, C), jnp.float32),
        scratch_types=[
            pltpu.VMEM((16, IDXROW), jnp.int32),
            pltpu.VMEM((IDXW, C), jnp.float32),
            pltpu.VMEM((IDXW, C), jnp.float32),
            pltpu.VMEM((2, POOL, 8, C), jnp.float32),
            pltpu.SemaphoreType.DMA,
            pltpu.SemaphoreType.DMA,
            pltpu.SemaphoreType.DMA,
            pltpu.SemaphoreType.DMA,
        ],
    )
    def sc_kernel(fmap_hbm, idx_hbm, out_hbm, idx_v, rows0, rows1,
                  out_v, gsem0, gsem1, ssem0, ssem1):
        wid = lax.axis_index("c") * 16 + lax.axis_index("s")
        q0 = (NGRP * wid) // NW
        q1 = (NGRP * (wid + 1)) // NW
        nq = q1 - q0
        bst = jnp.minimum(q0 // POOL, NBLK - 2)
        pltpu.sync_copy(idx_hbm.at[pl.ds(bst * 8, 16)], idx_v)

        def start_gather(q, rr, rows_v, gsem):
            lrow = (q // POOL - bst) * 8 + rr
            coff = (q % POOL) * IDXW
            pltpu.make_async_copy(
                fmap_hbm.at[idx_v.at[lrow, pl.ds(coff, IDXW)]],
                rows_v, gsem).start()

        def wait_gather(rows_v, gsem):
            pltpu.make_async_copy(fmap_hbm.at[pl.ds(0, IDXW)], rows_v, gsem).wait()

        def start_scatter(q, sb, ssem):
            b = q // POOL
            py = q % POOL
            smp = b // 16
            r0 = pl.multiple_of((b % 16) * 8, 8)
            pltpu.make_async_copy(
                out_v.at[sb], out_hbm.at[smp, py, :, pl.ds(r0, 8)],
                ssem).start()

        def wait_scatter(ssem):
            pltpu.make_async_copy(
                out_v.at[0], out_hbm.at[0, 0, :, pl.ds(0, 8)],
                ssem).wait()

        def compute(sb, rr, rows_v):
            def cell_body(px, carry):
                base = px * 9
                for grp in range(C // LANES):
                    sl = pl.ds(grp * LANES, LANES)
                    v = rows_v[base, sl]
                    for t in range(1, 9):
                        v = jnp.maximum(v, rows_v[base + t, sl])
                    out_v[sb, px, rr, sl] = v
                return carry

            lax.fori_loop(0, POOL, cell_body, 0)

        start_gather(q0, 0, rows0, gsem0)
        start_gather(q0, 1, rows1, gsem1)

        def group_body(i, carry):
            q = q0 + i
            sb = i % 2

            @pl.when((i >= 2) & (sb == 0))
            def _():
                wait_scatter(ssem0)

            @pl.when((i >= 2) & (sb == 1))
            def _():
                wait_scatter(ssem1)

            for rr in range(8):
                rows_v, gsem = (rows0, gsem0) if rr % 2 == 0 else (rows1, gsem1)
                wait_gather(rows_v, gsem)
                compute(sb, rr, rows_v)
                if rr < 6:
                    start_gather(q, rr + 2, rows_v, gsem)
                else:
                    @pl.when(i < nq - 1)
                    def _():
                        start_gather(q + 1, rr - 6, rows_v, gsem)

            @pl.when(sb == 0)
            def _():
                start_scatter(q, 0, ssem0)

            @pl.when(sb == 1)
            def _():
                start_scatter(q, 1, ssem1)

            return carry

        lax.fori_loop(0, nq, group_body, 0)
        wait_scatter(ssem0)
        wait_scatter(ssem1)

    return sc_kernel(fmap_flat, idx)


def _roi_pool_tc_kernel(rois_ref, fmap_ref, out_ref, tmp_ref):
    pg = pl.program_id(0)
    for rr in range(TCB):
        rg = NROI_SC + pg * TCB + rr
        y = rois_ref[rg, 0]
        x = rois_ref[rg, 1]
        h = rois_ref[rg, 2]
        w = rois_ref[rg, 3]

        ystep = h.astype(jnp.float32) / float(POOL)
        xstep = w.astype(jnp.float32) / float(POOL)

        col0 = jnp.minimum((x // 8) * 8, W - WIN)
        col0 = pl.multiple_of(col0, 8)
        lx = x - col0

        for py in range(POOL):
            ystart = (jnp.float32(py) * ystep).astype(jnp.int32)
            if py + 1 < POOL:
                yend = (jnp.float32(py + 1) * ystep).astype(jnp.int32)
            else:
                yend = h
            ysize = jnp.maximum(yend - ystart, 1)
            rows = fmap_ref[0, pl.ds(y + ystart, 3), pl.ds(col0, WIN), :]
            dy = lax.broadcasted_iota(jnp.int32, (3, 1, 1), 0)
            rows = jnp.where(dy < ysize, rows, NEG_INF)
            tmp_ref[:, py, :] = jnp.max(rows, axis=0)

        for px in range(POOL):
            xstart = (jnp.float32(px) * xstep).astype(jnp.int32)
            if px + 1 < POOL:
                xend = (jnp.float32(px + 1) * xstep).astype(jnp.int32)
            else:
                xend = w
            xsize = jnp.maximum(xend - xstart, 1)
            cols = tmp_ref[pl.ds(lx + xstart, 3), :, :]
            dx = lax.broadcasted_iota(jnp.int32, (3, 1, 1), 0)
            cols = jnp.where(dx < xsize, cols, NEG_INF)
            out_ref[0, :, px, rr, :] = jnp.max(cols, axis=0)


def _roi_pool_tc(x_maps, rois_flat):
    ngroups = NROI_TC // TCB
    goff = NROI_SC // TCB
    return pl.pallas_call(
        _roi_pool_tc_kernel,
        grid=(ngroups,),
        in_specs=[
            pl.BlockSpec((NROI, 4), lambda g: (0, 0),
                         memory_space=pltpu.SMEM),
            pl.BlockSpec((1, H, W, C), lambda g: ((goff + g) // 16, 0, 0, 0)),
        ],
        out_specs=pl.BlockSpec((1, POOL, POOL, TCB, C),
                               lambda g: ((goff + g) // 16, 0, 0,
                                          (goff + g) % 16, 0)),
        out_shape=jax.ShapeDtypeStruct((S, POOL, POOL, R, C), jnp.float32),
        scratch_shapes=[pltpu.VMEM((WIN, POOL, C), jnp.float32)],
    )(rois_flat, x_maps)


@jax.jit
def kernel(x_maps, x_rois):
    rois_flat = x_rois.reshape(NROI, 4)
    idx = _tap_indices(rois_flat[:NROI_SC])
    fmap_flat = x_maps.reshape(S * H * W, C)
    out_sc = _roi_pool_sc(fmap_flat, idx)
    out_tc = _roi_pool_tc(x_maps, rois_flat)
    out = lax.dynamic_update_slice(out_tc, out_sc[0:1], (0, 0, 0, 0, 0))
    out = lax.dynamic_update_slice(out, out_sc[1:2, :, :, :NROI_SC - R],
                                   (1, 0, 0, 0, 0))
    return out.transpose(0, 3, 1, 2, 4)
